# baseline (speedup 1.0000x reference)
import functools

import jax
import jax.numpy as jnp
from jax import lax
from jax.experimental import pallas as pl
from jax.experimental.pallas import tpu as pltpu
from jax.experimental.pallas import tpu_sc as plsc

_EMB = 102
_VOCABS = (6, 6, 3, 2, 4)
_DIVS = (144, 24, 8, 4, 1)
_NUM_COMB = 864
_NUM_PAD = 896
_OUT = 512
_BATCH = 16384

_NC = 2
_NS = 16
_NW = _NC * _NS
_SCROWS = 8192
_BPW = _SCROWS // _NW
_CHUNK = 64
_NBUF = 3
_PRIME = 2
_NCH = _BPW // _CHUNK
_LANES = 16


def _table_body(tg, tm, tt, tk, ts, w_ref, b2d, out_ref, pall_ref):
    tabs = (tg, tm, tt, tk, ts)
    w = w_ref[...]
    blocks = []
    for a in range(5):
        vocab = _VOCABS[a]
        row = [jnp.zeros((vocab, _EMB * a), jnp.float32)] if a else []
        row.append(tabs[a][...])
        if a < 4:
            row.append(jnp.zeros((vocab, _EMB * (4 - a)), jnp.float32))
        blocks.append(jnp.concatenate(row, axis=1) if len(row) > 1 else row[0])
    tall = jnp.concatenate(blocks, axis=0)
    pall = jnp.dot(tall, w, preferred_element_type=jnp.float32)
    pall_ref[...] = jnp.concatenate([pall, b2d[...]], axis=0)
    cid = lax.broadcasted_iota(jnp.int32, (_NUM_PAD, 1), 0)
    acc = jnp.broadcast_to(b2d[...], (_NUM_PAD, _OUT))
    off = 0
    for a in range(5):
        vocab, div = _VOCABS[a], _DIVS[a]
        sel = (cid // div) % vocab
        oh = (sel == lax.broadcasted_iota(jnp.int32, (_NUM_PAD, vocab), 1))
        p = lax.slice(pall, (off, 0), (off + vocab, _OUT))
        acc = acc + jnp.dot(oh.astype(jnp.float32), p,
                            preferred_element_type=jnp.float32)
        off += vocab
    out_ref[...] = acc


_build_table = pl.pallas_call(
    _table_body,
    out_shape=[jax.ShapeDtypeStruct((_NUM_PAD, _OUT), jnp.float32),
               jax.ShapeDtypeStruct((sum(_VOCABS) + 1, _OUT), jnp.float32)],
)


_TCBLK = 4096
_TCN = (_BATCH - _SCROWS) // _TCBLK


def _rows_body(o_any, gi, mi, ti, ki, si, pall_ref, out_ref):
    del o_any
    idxs = (gi, mi, ti, ki, si)
    rows = []
    for a in range(5):
        vocab = _VOCABS[a]
        idx_row = idxs[a][0]
        rows.append(
            (lax.broadcasted_iota(jnp.int32, (vocab, _TCBLK), 0) == idx_row)
            .astype(jnp.float32))
    rows.append(jnp.ones((1, _TCBLK), jnp.float32))
    oht = jnp.concatenate(rows, axis=0)
    out_ref[...] = lax.dot_general(
        oht, pall_ref[...], (((0,), (0,)), ((), ())),
        preferred_element_type=jnp.float32)


_fill_rows = pl.pallas_call(
    _rows_body,
    grid=(_TCN,),
    in_specs=[
        pl.BlockSpec(memory_space=pl.ANY),
        pl.BlockSpec((1, 1, _TCBLK), lambda i: (_SCROWS // _TCBLK + i, 0, 0)),
        pl.BlockSpec((1, 1, _TCBLK), lambda i: (_SCROWS // _TCBLK + i, 0, 0)),
        pl.BlockSpec((1, 1, _TCBLK), lambda i: (_SCROWS // _TCBLK + i, 0, 0)),
        pl.BlockSpec((1, 1, _TCBLK), lambda i: (_SCROWS // _TCBLK + i, 0, 0)),
        pl.BlockSpec((1, 1, _TCBLK), lambda i: (_SCROWS // _TCBLK + i, 0, 0)),
        pl.BlockSpec((sum(_VOCABS) + 1, _OUT), lambda i: (0, 0)),
    ],
    out_specs=pl.BlockSpec((_TCBLK, _OUT), lambda i: (_SCROWS // _TCBLK + i, 0)),
    out_shape=jax.ShapeDtypeStruct((_BATCH, _OUT), jnp.float32),
    input_output_aliases={0: 0},
)


def _gather_body(table_hbm, g_hbm, m_hbm, t_hbm, k_hbm, s_hbm, out_hbm,
                 gv, mv, tv, kv, sv, cidx, bufs_ref, gsems, wsems, isems):
    wid = lax.axis_index("s") * _NC + lax.axis_index("c")
    base = wid * _BPW
    ih = [pltpu.async_copy(src_ref.at[pl.ds(base, _BPW)], dst_ref, isems.at[i])
          for i, (src_ref, dst_ref) in enumerate(
              [(g_hbm, gv), (m_hbm, mv), (t_hbm, tv), (k_hbm, kv), (s_hbm, sv)])]
    for h in ih:
        h.wait()

    def cbody(j, carry):
        off = j * _LANES
        c = (gv[pl.ds(off, _LANES)] * _DIVS[0]
             + mv[pl.ds(off, _LANES)] * _DIVS[1]
             + tv[pl.ds(off, _LANES)] * _DIVS[2]
             + kv[pl.ds(off, _LANES)] * _DIVS[3]
             + sv[pl.ds(off, _LANES)])
        cidx[pl.ds(off, _LANES)] = c
        return carry

    lax.fori_loop(0, _BPW // _LANES, cbody, 0)

    def issue_gather(c):
        b = c % _NBUF
        return pltpu.async_copy(
            table_hbm.at[cidx.at[pl.ds(c * _CHUNK, _CHUNK)]],
            bufs_ref.at[b], gsems.at[b])

    def issue_write(c):
        b = c % _NBUF
        return pltpu.async_copy(
            bufs_ref.at[b], out_hbm.at[pl.ds(base + c * _CHUNK, _CHUNK)],
            wsems.at[b])

    gh = {c: issue_gather(c) for c in range(_PRIME)}
    wh = {}
    waited = set()
    for c in range(_NCH):
        gh[c].wait()
        wh[c] = issue_write(c)
        nxt = c + _PRIME
        if nxt < _NCH:
            prev = nxt - _NBUF
            if prev >= 0:
                wh[prev].wait()
                waited.add(prev)
            gh[nxt] = issue_gather(nxt)
    for c in range(_NCH):
        if c not in waited:
            wh[c].wait()


@functools.lru_cache(maxsize=None)
def _make_gather():
    return pl.kernel(
        _gather_body,
        out_type=jax.ShapeDtypeStruct((_BATCH, _OUT), jnp.float32),
        mesh=plsc.VectorSubcoreMesh(core_axis_name="c", subcore_axis_name="s"),
        scratch_types=[
            pltpu.VMEM((_BPW,), jnp.int32),
            pltpu.VMEM((_BPW,), jnp.int32),
            pltpu.VMEM((_BPW,), jnp.int32),
            pltpu.VMEM((_BPW,), jnp.int32),
            pltpu.VMEM((_BPW,), jnp.int32),
            pltpu.VMEM((_BPW,), jnp.int32),
            pltpu.VMEM((_NBUF, _CHUNK, _OUT), jnp.float32),
            pltpu.SemaphoreType.DMA((_NBUF,)),
            pltpu.SemaphoreType.DMA((_NBUF,)),
            pltpu.SemaphoreType.DMA((5,)),
        ],
    )


def kernel(genre, mood, tempo, key_mode, time_signature,
           emb_genre, emb_mood, emb_tempo, emb_key_mode, emb_time_signature,
           W, b):
    table, pall = _build_table(emb_genre, emb_mood, emb_tempo, emb_key_mode,
                               emb_time_signature, W, b.reshape(1, _OUT))
    idxs = [jnp.asarray(x, jnp.int32)
            for x in (genre, mood, tempo, key_mode, time_signature)]
    o1 = _make_gather()(table, *idxs)
    tc_idx = [jnp.reshape(x, (_BATCH // _TCBLK, 1, _TCBLK)) for x in idxs]
    return _fill_rows(o1, *tc_idx, pall)


# speedup vs baseline: 1.0723x; 1.0723x over previous
import functools

import jax
import jax.numpy as jnp
from jax import lax
from jax.experimental import pallas as pl
from jax.experimental.pallas import tpu as pltpu
from jax.experimental.pallas import tpu_sc as plsc

_EMB = 102
_VOCABS = (6, 6, 3, 2, 4)
_DIVS = (144, 24, 8, 4, 1)
_NUM_COMB = 864
_NUM_PAD = 896
_OUT = 512
_BATCH = 16384

_NC = 2
_NS = 16
_NW = _NC * _NS
_SCROWS = 6144
_BPW = _SCROWS // _NW
_CHUNK = 64
_NBUF = 3
_PRIME = 2
_NCH = _BPW // _CHUNK
_LANES = 16


def _table_body(tg, tm, tt, tk, ts, w_ref, b2d, out_ref, pall_ref):
    tabs = (tg, tm, tt, tk, ts)
    w = w_ref[...]
    blocks = []
    for a in range(5):
        vocab = _VOCABS[a]
        row = [jnp.zeros((vocab, _EMB * a), jnp.float32)] if a else []
        row.append(tabs[a][...])
        if a < 4:
            row.append(jnp.zeros((vocab, _EMB * (4 - a)), jnp.float32))
        blocks.append(jnp.concatenate(row, axis=1) if len(row) > 1 else row[0])
    tall = jnp.concatenate(blocks, axis=0)
    pall = jnp.dot(tall, w, preferred_element_type=jnp.float32)
    pall_ref[...] = jnp.concatenate([pall, b2d[...]], axis=0)
    cid = lax.broadcasted_iota(jnp.int32, (_NUM_PAD, 1), 0)
    acc = jnp.broadcast_to(b2d[...], (_NUM_PAD, _OUT))
    off = 0
    for a in range(5):
        vocab, div = _VOCABS[a], _DIVS[a]
        sel = (cid // div) % vocab
        oh = (sel == lax.broadcasted_iota(jnp.int32, (_NUM_PAD, vocab), 1))
        p = lax.slice(pall, (off, 0), (off + vocab, _OUT))
        acc = acc + jnp.dot(oh.astype(jnp.float32), p,
                            preferred_element_type=jnp.float32)
        off += vocab
    out_ref[...] = acc


_build_table = pl.pallas_call(
    _table_body,
    out_shape=[jax.ShapeDtypeStruct((_NUM_PAD, _OUT), jnp.float32),
               jax.ShapeDtypeStruct((sum(_VOCABS) + 1, _OUT), jnp.float32)],
)


_TCBLK = 2048
_TCN = (_BATCH - _SCROWS) // _TCBLK


def _rows_body(o_any, gi, mi, ti, ki, si, pall_ref, out_ref):
    del o_any
    idxs = (gi, mi, ti, ki, si)
    rows = []
    for a in range(5):
        vocab = _VOCABS[a]
        idx_row = idxs[a][0]
        rows.append(
            (lax.broadcasted_iota(jnp.int32, (vocab, _TCBLK), 0) == idx_row)
            .astype(jnp.float32))
    rows.append(jnp.ones((1, _TCBLK), jnp.float32))
    oht = jnp.concatenate(rows, axis=0)
    out_ref[...] = lax.dot_general(
        oht, pall_ref[...], (((0,), (0,)), ((), ())),
        preferred_element_type=jnp.float32)


_fill_rows = pl.pallas_call(
    _rows_body,
    grid=(_TCN,),
    in_specs=[
        pl.BlockSpec(memory_space=pl.ANY),
        pl.BlockSpec((1, 1, _TCBLK), lambda i: (_SCROWS // _TCBLK + i, 0, 0)),
        pl.BlockSpec((1, 1, _TCBLK), lambda i: (_SCROWS // _TCBLK + i, 0, 0)),
        pl.BlockSpec((1, 1, _TCBLK), lambda i: (_SCROWS // _TCBLK + i, 0, 0)),
        pl.BlockSpec((1, 1, _TCBLK), lambda i: (_SCROWS // _TCBLK + i, 0, 0)),
        pl.BlockSpec((1, 1, _TCBLK), lambda i: (_SCROWS // _TCBLK + i, 0, 0)),
        pl.BlockSpec((sum(_VOCABS) + 1, _OUT), lambda i: (0, 0)),
    ],
    out_specs=pl.BlockSpec((_TCBLK, _OUT), lambda i: (_SCROWS // _TCBLK + i, 0)),
    out_shape=jax.ShapeDtypeStruct((_BATCH, _OUT), jnp.float32),
    input_output_aliases={0: 0},
)


def _gather_body(table_hbm, g_hbm, m_hbm, t_hbm, k_hbm, s_hbm, out_hbm,
                 gv, mv, tv, kv, sv, cidx, bufs_ref, gsems, wsems, isems):
    wid = lax.axis_index("s") * _NC + lax.axis_index("c")
    base = wid * _BPW
    ih = [pltpu.async_copy(src_ref.at[pl.ds(base, _BPW)], dst_ref, isems.at[i])
          for i, (src_ref, dst_ref) in enumerate(
              [(g_hbm, gv), (m_hbm, mv), (t_hbm, tv), (k_hbm, kv), (s_hbm, sv)])]
    for h in ih:
        h.wait()

    def cbody(j, carry):
        off = j * _LANES
        c = (gv[pl.ds(off, _LANES)] * _DIVS[0]
             + mv[pl.ds(off, _LANES)] * _DIVS[1]
             + tv[pl.ds(off, _LANES)] * _DIVS[2]
             + kv[pl.ds(off, _LANES)] * _DIVS[3]
             + sv[pl.ds(off, _LANES)])
        cidx[pl.ds(off, _LANES)] = c
        return carry

    lax.fori_loop(0, _BPW // _LANES, cbody, 0)

    def issue_gather(c):
        b = c % _NBUF
        return pltpu.async_copy(
            table_hbm.at[cidx.at[pl.ds(c * _CHUNK, _CHUNK)]],
            bufs_ref.at[b], gsems.at[b])

    def issue_write(c):
        b = c % _NBUF
        return pltpu.async_copy(
            bufs_ref.at[b], out_hbm.at[pl.ds(base + c * _CHUNK, _CHUNK)],
            wsems.at[b])

    gh = {c: issue_gather(c) for c in range(_PRIME)}
    wh = {}
    waited = set()
    for c in range(_NCH):
        gh[c].wait()
        wh[c] = issue_write(c)
        nxt = c + _PRIME
        if nxt < _NCH:
            prev = nxt - _NBUF
            if prev >= 0:
                wh[prev].wait()
                waited.add(prev)
            gh[nxt] = issue_gather(nxt)
    for c in range(_NCH):
        if c not in waited:
            wh[c].wait()


@functools.lru_cache(maxsize=None)
def _make_gather():
    return pl.kernel(
        _gather_body,
        out_type=jax.ShapeDtypeStruct((_BATCH, _OUT), jnp.float32),
        mesh=plsc.VectorSubcoreMesh(core_axis_name="c", subcore_axis_name="s"),
        scratch_types=[
            pltpu.VMEM((_BPW,), jnp.int32),
            pltpu.VMEM((_BPW,), jnp.int32),
            pltpu.VMEM((_BPW,), jnp.int32),
            pltpu.VMEM((_BPW,), jnp.int32),
            pltpu.VMEM((_BPW,), jnp.int32),
            pltpu.VMEM((_BPW,), jnp.int32),
            pltpu.VMEM((_NBUF, _CHUNK, _OUT), jnp.float32),
            pltpu.SemaphoreType.DMA((_NBUF,)),
            pltpu.SemaphoreType.DMA((_NBUF,)),
            pltpu.SemaphoreType.DMA((5,)),
        ],
    )


def kernel(genre, mood, tempo, key_mode, time_signature,
           emb_genre, emb_mood, emb_tempo, emb_key_mode, emb_time_signature,
           W, b):
    table, pall = _build_table(emb_genre, emb_mood, emb_tempo, emb_key_mode,
                               emb_time_signature, W, b.reshape(1, _OUT))
    idxs = [jnp.asarray(x, jnp.int32)
            for x in (genre, mood, tempo, key_mode, time_signature)]
    o1 = _make_gather()(table, *idxs)
    tc_idx = [jnp.reshape(x, (_BATCH // _TCBLK, 1, _TCBLK)) for x in idxs]
    return _fill_rows(o1, *tc_idx, pall)
